# Initial kernel scaffold; baseline (speedup 1.0000x reference)
import jax
import jax.numpy as jnp
from jax.experimental import pallas as pl


def kernel(input_ids, tok_struct_vec, word_emb, pos_emb, type_emb, ln_gamma, ln_beta):
    raise NotImplementedError("write your pallas kernel here")


# speedup vs baseline: 5.5603x; 5.5603x over previous
import functools

import jax
import jax.numpy as jnp
from jax import lax
from jax.experimental import pallas as pl
from jax.experimental.pallas import tpu as pltpu
from jax.experimental.pallas import tpu_sc as plsc

B, L, H = 1024, 200, 128
V, P, T = 100000, 512, 2
N = B * L
EPS = 1e-12

_INFO = plsc.get_sparse_core_info()
NC = _INFO.num_cores
NS = _INFO.num_subcores
NW = NC * NS
TOK_PER_W = N // NW
CH = 64
NCHUNK = TOK_PER_W // CH
NH = H // 16


def _emb_ln_kernel(ids_hbm, p0_hbm, p1_hbm, p2_hbm, word_hbm, pos_hbm,
                   type_hbm, gb_hbm, out_hbm,
                   static_v, idw_v, id0_v, id1_v, id2_v,
                   wbuf, b0, b1, b2, obuf, gb_v, sem):
    wid = lax.axis_index("s") * NC + lax.axis_index("c")
    tok_base = wid * TOK_PER_W

    pltpu.sync_copy(pos_hbm.at[pl.ds(0, L)], static_v)
    pltpu.sync_copy(type_hbm.at[pl.ds(0, 1)], gb_v.at[pl.ds(0, 1)])
    pltpu.sync_copy(gb_hbm, gb_v.at[pl.ds(1, 2)])

    def add_t0(r, _):
        for c in range(NH):
            col = pl.ds(c * 16, 16)
            static_v[r, col] = static_v[r, col] + gb_v[0, col]
        return 0
    lax.fori_loop(0, L, add_t0, 0)

    gamma = [gb_v[1, pl.ds(c * 16, 16)] for c in range(NH)]
    beta = [gb_v[2, pl.ds(c * 16, 16)] for c in range(NH)]

    lanes = lax.iota(jnp.int32, 16)
    shuf_idx = [lanes ^ k for k in (1, 2, 4, 8)]
    dnums = lax.GatherDimensionNumbers(
        offset_dims=(), collapsed_slice_dims=(0,), start_index_map=(0,))

    def allsum(x):
        for idx in shuf_idx:
            x = x + lax.gather(
                x, idx[:, None], dimension_numbers=dnums, slice_sizes=(1,),
                mode=lax.GatherScatterMode.PROMISE_IN_BOUNDS)
        return x

    def chunk_body(g, _):
        base = tok_base + g * CH
        base_l = (g * CH) % L

        pltpu.sync_copy(ids_hbm.at[pl.ds(base, CH)], idw_v)
        pltpu.sync_copy(p0_hbm.at[pl.ds(base, CH)], id0_v)
        pltpu.sync_copy(p1_hbm.at[pl.ds(base, CH)], id1_v)
        pltpu.sync_copy(p2_hbm.at[pl.ds(base, CH)], id2_v)

        h0 = pltpu.async_copy(word_hbm.at[idw_v], wbuf, sem)
        h1 = pltpu.async_copy(pos_hbm.at[id0_v], b0, sem)
        h2 = pltpu.async_copy(pos_hbm.at[id1_v], b1, sem)
        h3 = pltpu.async_copy(pos_hbm.at[id2_v], b2, sem)
        h0.wait(); h1.wait(); h2.wait(); h3.wait()

        def tok_body(t, _):
            l = base_l + t
            l = jnp.where(l >= L, l - L, l)
            e = []
            for c in range(NH):
                col = pl.ds(c * 16, 16)
                v = (wbuf[t, col] + b0[t, col] + b1[t, col] + b2[t, col]
                     + static_v[l, col])
                e.append(v)
            tot = e[0]
            sq = e[0] * e[0]
            for c in range(1, NH):
                tot = tot + e[c]
                sq = sq + e[c] * e[c]
            meanv = allsum(tot) * (1.0 / H)
            vv = allsum(sq) * (1.0 / H) - meanv * meanv + EPS
            yi = 0x5F3759DF - (lax.bitcast_convert_type(vv, jnp.int32) >> 1)
            y = lax.bitcast_convert_type(yi, jnp.float32)
            half = vv * 0.5
            for _ in range(3):
                y = y * (1.5 - half * y * y)
            for c in range(NH):
                col = pl.ds(c * 16, 16)
                obuf[t, col] = (e[c] - meanv) * y * gamma[c] + beta[c]
            return 0
        lax.fori_loop(0, CH, tok_body, 0)

        pltpu.sync_copy(obuf, out_hbm.at[pl.ds(base, CH)])
        return 0

    lax.fori_loop(0, NCHUNK, chunk_body, 0)


@jax.jit
def _run(ids, p0, p1, p2, word_emb, pos_emb, type_emb, gb):
    mesh = plsc.VectorSubcoreMesh(core_axis_name="c", subcore_axis_name="s")
    k = functools.partial(
        pl.kernel,
        mesh=mesh,
        out_type=jax.ShapeDtypeStruct((N, H), jnp.float32),
        scratch_types=[
            pltpu.VMEM((L, H), jnp.float32),
            pltpu.VMEM((CH,), jnp.int32),
            pltpu.VMEM((CH,), jnp.int32),
            pltpu.VMEM((CH,), jnp.int32),
            pltpu.VMEM((CH,), jnp.int32),
            pltpu.VMEM((CH, H), jnp.float32),
            pltpu.VMEM((CH, H), jnp.float32),
            pltpu.VMEM((CH, H), jnp.float32),
            pltpu.VMEM((CH, H), jnp.float32),
            pltpu.VMEM((CH, H), jnp.float32),
            pltpu.VMEM((3, H), jnp.float32),
            pltpu.SemaphoreType.DMA,
        ],
    )(_emb_ln_kernel)
    return k(ids, p0, p1, p2, word_emb, pos_emb, type_emb, gb)


def kernel(input_ids, tok_struct_vec, word_emb, pos_emb, type_emb,
           ln_gamma, ln_beta):
    ids = input_ids.reshape(-1).astype(jnp.int32)
    p0 = tok_struct_vec[:, :, 0].reshape(-1).astype(jnp.int32)
    p1 = tok_struct_vec[:, :, 1].reshape(-1).astype(jnp.int32)
    p2 = tok_struct_vec[:, :, 2].reshape(-1).astype(jnp.int32)
    gb = jnp.stack([ln_gamma, ln_beta]).astype(jnp.float32)
    out = _run(ids, p0, p1, p2, word_emb.astype(jnp.float32),
               pos_emb.astype(jnp.float32), type_emb.astype(jnp.float32), gb)
    return out.reshape(B, L, H)
